# baseline (speedup 1.0000x reference)
import dataclasses
import functools

import jax
import jax.numpy as jnp
from jax import lax
from jax.experimental import pallas as pl
from jax.experimental.pallas import tpu as pltpu
from jax.experimental.pallas import tpu_sc as plsc

NUM_EMOTIONS = 1000
EMO_DIM = 64
BATCH = 16384

_NC = 2
_NS = 16
_L = 16
_NW = _NC * _NS
_BPW = BATCH // _NW
_JPW = _BPW // 128
_TD = EMO_DIM // 8

_mesh = plsc.VectorSubcoreMesh(core_axis_name="c", subcore_axis_name="s")

_cp = pltpu.CompilerParams()
if "needs_layout_passes" in pltpu.CompilerParams.__dataclass_fields__:
    _cp = dataclasses.replace(_cp, needs_layout_passes=False)
if "use_tc_tiling_on_sc" in pltpu.CompilerParams.__dataclass_fields__:
    _cp = dataclasses.replace(_cp, use_tc_tiling_on_sc=False)


@jax.jit
def _emotion_encode(emo_id, strength, table_flat):
    @functools.partial(
        pl.kernel,
        out_type=jax.ShapeDtypeStruct((BATCH // 2, 2 * EMO_DIM), jnp.float32),
        mesh=_mesh,
        compiler_params=_cp,
        scratch_types=[
            pltpu.VMEM((NUM_EMOTIONS * EMO_DIM,), jnp.float32),
            pltpu.VMEM((_BPW,), jnp.int32),
            pltpu.VMEM((_BPW,), jnp.float32),
            pltpu.VMEM((EMO_DIM, 128), jnp.float32),
        ],
    )
    def k(emo_hbm, str_hbm, tab_hbm, out_hbm, tab_v, idx_v, str_v, trows_v):
        wid = lax.axis_index("s") * _NC + lax.axis_index("c")
        base = wid * _BPW
        pltpu.sync_copy(emo_hbm.at[pl.ds(base, _BPW)], idx_v)
        pltpu.sync_copy(str_hbm.at[pl.ds(base, _BPW)], str_v)
        pltpu.sync_copy(tab_hbm, tab_v)

        @pl.loop(0, _JPW)
        def _(jl):
            @pl.loop(0, 128 // _L)
            def _(bg):
                o = jl * 128 + bg * _L
                e = idx_v[pl.ds(o, _L)]
                s = str_v[pl.ds(o, _L)]
                ebase = e * EMO_DIM
                for d in range(EMO_DIM):
                    v = plsc.load_gather(tab_v, [ebase + d])
                    trows_v[d, pl.ds(bg * _L, _L)] = v * s

            jg = wid * _JPW + jl
            for i in range(_TD):
                pltpu.sync_copy(
                    trows_v.at[pl.ds(8 * i, 8)],
                    out_hbm.at[pl.ds((i * 128 + jg) * 8, 8)],
                )

    return k(emo_id, strength, table_flat)


def kernel(emo_id, strength, table):
    w = _emotion_encode(
        emo_id.astype(jnp.int32), strength, table.reshape(-1)
    )
    return (
        w.reshape(_TD, 128, 8, 128)
        .transpose(1, 3, 0, 2)
        .reshape(BATCH, EMO_DIM)
    )


# speedup vs baseline: 1.3289x; 1.3289x over previous
import dataclasses
import functools

import jax
import jax.numpy as jnp
from jax import lax
from jax.experimental import pallas as pl
from jax.experimental.pallas import tpu as pltpu
from jax.experimental.pallas import tpu_sc as plsc

NUM_EMOTIONS = 1000
EMO_DIM = 64
BATCH = 16384

_NC = 2
_NS = 16
_L = 16
_NW = _NC * _NS
_BPW = BATCH // _NW
_JPW = _BPW // 128
_TD = EMO_DIM // 8

_mesh = plsc.VectorSubcoreMesh(core_axis_name="c", subcore_axis_name="s")

_cp = pltpu.CompilerParams()
if "needs_layout_passes" in pltpu.CompilerParams.__dataclass_fields__:
    _cp = dataclasses.replace(_cp, needs_layout_passes=False)
if "use_tc_tiling_on_sc" in pltpu.CompilerParams.__dataclass_fields__:
    _cp = dataclasses.replace(_cp, use_tc_tiling_on_sc=False)


@jax.jit
def _emotion_encode(emo_id, strength, table_flat):
    @functools.partial(
        pl.kernel,
        out_type=jax.ShapeDtypeStruct((BATCH // 2, 2 * EMO_DIM), jnp.float32),
        mesh=_mesh,
        compiler_params=_cp,
        scratch_types=[
            pltpu.VMEM((NUM_EMOTIONS * EMO_DIM,), jnp.float32),
            pltpu.VMEM((_BPW,), jnp.int32),
            pltpu.VMEM((_BPW,), jnp.float32),
            pltpu.VMEM((EMO_DIM, 128), jnp.float32),
        ],
    )
    def k(emo_hbm, str_hbm, tab_hbm, out_hbm, tab_v, idx_v, str_v, trows_v):
        wid = lax.axis_index("s") * _NC + lax.axis_index("c")
        base = wid * _BPW
        pltpu.sync_copy(emo_hbm.at[pl.ds(base, _BPW)], idx_v)
        pltpu.sync_copy(str_hbm.at[pl.ds(base, _BPW)], str_v)
        pltpu.sync_copy(tab_hbm, tab_v)

        @pl.loop(0, _JPW)
        def _(jl):
            @plsc.parallel_loop(0, 128 // _L)
            def _(bg):
                o = jl * 128 + bg * _L
                e = idx_v[pl.ds(o, _L)]
                s = str_v[pl.ds(o, _L)]
                ebase = e * EMO_DIM

                @plsc.parallel_loop(0, EMO_DIM, unroll=8)
                def _(d):
                    v = plsc.load_gather(tab_v, [ebase + d])
                    trows_v[d, pl.ds(bg * _L, _L)] = v * s

            jg = wid * _JPW + jl
            for i in range(_TD):
                pltpu.sync_copy(
                    trows_v.at[pl.ds(8 * i, 8)],
                    out_hbm.at[pl.ds((i * 128 + jg) * 8, 8)],
                )

    return k(emo_id, strength, table_flat)


def kernel(emo_id, strength, table):
    w = _emotion_encode(
        emo_id.astype(jnp.int32), strength, table.reshape(-1)
    )
    return (
        w.reshape(_TD, 128, 8, 128)
        .transpose(1, 3, 0, 2)
        .reshape(BATCH, EMO_DIM)
    )


# speedup vs baseline: 1.4119x; 1.0625x over previous
import dataclasses
import functools

import jax
import jax.numpy as jnp
from jax import lax
from jax.experimental import pallas as pl
from jax.experimental.pallas import tpu as pltpu
from jax.experimental.pallas import tpu_sc as plsc

NUM_EMOTIONS = 1000
EMO_DIM = 64
BATCH = 16384

_NC = 2
_NS = 16
_L = 16
_NW = _NC * _NS
_BPW = BATCH // _NW
_JPW = _BPW // 128
_TD = EMO_DIM // 8

_mesh = plsc.VectorSubcoreMesh(core_axis_name="c", subcore_axis_name="s")

_cp = pltpu.CompilerParams()
if "needs_layout_passes" in pltpu.CompilerParams.__dataclass_fields__:
    _cp = dataclasses.replace(_cp, needs_layout_passes=False)
if "use_tc_tiling_on_sc" in pltpu.CompilerParams.__dataclass_fields__:
    _cp = dataclasses.replace(_cp, use_tc_tiling_on_sc=False)


@jax.jit
def _emotion_encode(emo_id, strength, table_flat):
    @functools.partial(
        pl.kernel,
        out_type=jax.ShapeDtypeStruct((BATCH // 2, 2 * EMO_DIM), jnp.float32),
        mesh=_mesh,
        compiler_params=_cp,
        scratch_types=[
            pltpu.VMEM((NUM_EMOTIONS * EMO_DIM,), jnp.float32),
            pltpu.VMEM((_BPW,), jnp.int32),
            pltpu.VMEM((_BPW,), jnp.float32),
        ]
        + [pltpu.VMEM((EMO_DIM, 128), jnp.float32) for _ in range(_JPW)]
        + [pltpu.SemaphoreType.DMA, pltpu.SemaphoreType.DMA],
    )
    def k(emo_hbm, str_hbm, tab_hbm, out_hbm, tab_v, idx_v, str_v,
          t0, t1, t2, t3, sem_t, sem_o):
        wid = lax.axis_index("s") * _NC + lax.axis_index("c")
        base = wid * _BPW
        tab_copy = pltpu.async_copy(tab_hbm, tab_v, sem_t)
        pltpu.sync_copy(emo_hbm.at[pl.ds(base, _BPW)], idx_v)
        pltpu.sync_copy(str_hbm.at[pl.ds(base, _BPW)], str_v)
        tab_copy.wait()

        out_copies = []
        for jl, trows_v in enumerate((t0, t1, t2, t3)):
            @plsc.parallel_loop(0, 128 // _L)
            def _(bg, jl=jl, trows_v=trows_v):
                o = jl * 128 + bg * _L
                e = idx_v[pl.ds(o, _L)]
                s = str_v[pl.ds(o, _L)]
                ebase = e * EMO_DIM

                @plsc.parallel_loop(0, EMO_DIM, unroll=8)
                def _(d):
                    v = plsc.load_gather(tab_v, [ebase + d])
                    trows_v[d, pl.ds(bg * _L, _L)] = v * s

            jg = wid * _JPW + jl
            for i in range(_TD):
                out_copies.append(pltpu.async_copy(
                    trows_v.at[pl.ds(8 * i, 8)],
                    out_hbm.at[pl.ds((i * 128 + jg) * 8, 8)],
                    sem_o,
                ))
        for c in out_copies:
            c.wait()

    return k(emo_id, strength, table_flat)


def kernel(emo_id, strength, table):
    w = _emotion_encode(
        emo_id.astype(jnp.int32), strength, table.reshape(-1)
    )
    return (
        w.reshape(_TD, 128, 8, 128)
        .transpose(1, 3, 0, 2)
        .reshape(BATCH, EMO_DIM)
    )


# speedup vs baseline: 1.4515x; 1.0281x over previous
import dataclasses
import functools

import jax
import jax.numpy as jnp
from jax import lax
from jax.experimental import pallas as pl
from jax.experimental.pallas import tpu as pltpu
from jax.experimental.pallas import tpu_sc as plsc

NUM_EMOTIONS = 1000
EMO_DIM = 64
BATCH = 16384

_NC = 2
_NS = 16
_L = 16
_NW = _NC * _NS
_BPW = BATCH // _NW
_JPW = _BPW // 128
_TD = EMO_DIM // 8

_mesh = plsc.VectorSubcoreMesh(core_axis_name="c", subcore_axis_name="s")

_cp = pltpu.CompilerParams()
if "needs_layout_passes" in pltpu.CompilerParams.__dataclass_fields__:
    _cp = dataclasses.replace(_cp, needs_layout_passes=False)
if "use_tc_tiling_on_sc" in pltpu.CompilerParams.__dataclass_fields__:
    _cp = dataclasses.replace(_cp, use_tc_tiling_on_sc=False)


@jax.jit
def _emotion_encode(emo_id, strength, table):
    @functools.partial(
        pl.kernel,
        out_type=jax.ShapeDtypeStruct((BATCH // 2, 2 * EMO_DIM), jnp.float32),
        mesh=_mesh,
        compiler_params=_cp,
        scratch_types=[
            pltpu.VMEM((_BPW,), jnp.int32),
            pltpu.VMEM((_BPW,), jnp.float32),
            pltpu.VMEM((_BPW, EMO_DIM), jnp.float32),
        ]
        + [pltpu.VMEM((EMO_DIM, 128), jnp.float32) for _ in range(_JPW)]
        + [pltpu.SemaphoreType.DMA, pltpu.SemaphoreType.DMA],
    )
    def k(emo_hbm, str_hbm, tab_hbm, out_hbm, idx_v, str_v, grows_v,
          t0, t1, t2, t3, sem_g, sem_o):
        wid = lax.axis_index("s") * _NC + lax.axis_index("c")
        base = wid * _BPW
        pltpu.sync_copy(emo_hbm.at[pl.ds(base, _BPW)], idx_v)
        pltpu.sync_copy(str_hbm.at[pl.ds(base, _BPW)], str_v)
        pltpu.async_copy(tab_hbm.at[idx_v], grows_v, sem_g).wait()

        lane = lax.iota(jnp.int32, _L)
        out_copies = []
        for jl, trows_v in enumerate((t0, t1, t2, t3)):
            @plsc.parallel_loop(0, 128 // _L)
            def _(bg, jl=jl, trows_v=trows_v):
                o = jl * 128 + bg * _L
                s = str_v[pl.ds(o, _L)]
                rows = o + lane

                @plsc.parallel_loop(0, EMO_DIM, unroll=8)
                def _(d):
                    dv = jnp.broadcast_to(d, (_L,)).astype(jnp.int32)
                    v = plsc.load_gather(grows_v, [rows, dv])
                    trows_v[d, pl.ds(bg * _L, _L)] = v * s

            jg = wid * _JPW + jl
            for i in range(_TD):
                out_copies.append(pltpu.async_copy(
                    trows_v.at[pl.ds(8 * i, 8)],
                    out_hbm.at[pl.ds((i * 128 + jg) * 8, 8)],
                    sem_o,
                ))
        for c in out_copies:
            c.wait()

    return k(emo_id, strength, table)


def kernel(emo_id, strength, table):
    w = _emotion_encode(emo_id.astype(jnp.int32), strength, table)
    return (
        w.reshape(_TD, 128, 8, 128)
        .transpose(1, 3, 0, 2)
        .reshape(BATCH, EMO_DIM)
    )


# speedup vs baseline: 1.9605x; 1.3507x over previous
import dataclasses
import functools

import jax
import jax.numpy as jnp
from jax import lax
from jax.experimental import pallas as pl
from jax.experimental.pallas import tpu as pltpu
from jax.experimental.pallas import tpu_sc as plsc

NUM_EMOTIONS = 1000
EMO_DIM = 64
_PITCH = EMO_DIM + 1
BATCH = 16384

_NC = 2
_NS = 16
_L = 16
_NW = _NC * _NS
_BPW = BATCH // _NW
_JPW = _BPW // 128
_TD = EMO_DIM // 8

_mesh = plsc.VectorSubcoreMesh(core_axis_name="c", subcore_axis_name="s")

_cp = pltpu.CompilerParams()
if "needs_layout_passes" in pltpu.CompilerParams.__dataclass_fields__:
    _cp = dataclasses.replace(_cp, needs_layout_passes=False)
if "use_tc_tiling_on_sc" in pltpu.CompilerParams.__dataclass_fields__:
    _cp = dataclasses.replace(_cp, use_tc_tiling_on_sc=False)


@jax.jit
def _emotion_encode(emo_id, strength, table_flat):
    @functools.partial(
        pl.kernel,
        out_type=jax.ShapeDtypeStruct((BATCH // 2, 2 * EMO_DIM), jnp.float32),
        mesh=_mesh,
        compiler_params=_cp,
        scratch_types=[
            pltpu.VMEM((NUM_EMOTIONS * _PITCH,), jnp.float32),
            pltpu.VMEM((_BPW,), jnp.int32),
            pltpu.VMEM((_BPW,), jnp.float32),
        ]
        + [pltpu.VMEM((EMO_DIM, 128), jnp.float32) for _ in range(_JPW)]
        + [pltpu.SemaphoreType.DMA, pltpu.SemaphoreType.DMA],
    )
    def k(emo_hbm, str_hbm, tab_hbm, out_hbm, tab_v, idx_v, str_v,
          t0, t1, t2, t3, sem_t, sem_o):
        wid = lax.axis_index("s") * _NC + lax.axis_index("c")
        base = wid * _BPW
        tab_copy = pltpu.async_copy(tab_hbm, tab_v, sem_t)
        pltpu.sync_copy(emo_hbm.at[pl.ds(base, _BPW)], idx_v)
        pltpu.sync_copy(str_hbm.at[pl.ds(base, _BPW)], str_v)
        tab_copy.wait()

        out_copies = []
        for jl, trows_v in enumerate((t0, t1, t2, t3)):
            @plsc.parallel_loop(0, 128 // _L)
            def _(bg, jl=jl, trows_v=trows_v):
                o = jl * 128 + bg * _L
                e = idx_v[pl.ds(o, _L)]
                s = str_v[pl.ds(o, _L)]
                ebase = e * _PITCH

                @plsc.parallel_loop(0, EMO_DIM, unroll=8)
                def _(d):
                    v = plsc.load_gather(tab_v, [ebase + d])
                    trows_v[d, pl.ds(bg * _L, _L)] = v * s

            jg = wid * _JPW + jl
            for i in range(_TD):
                out_copies.append(pltpu.async_copy(
                    trows_v.at[pl.ds(8 * i, 8)],
                    out_hbm.at[pl.ds((i * 128 + jg) * 8, 8)],
                    sem_o,
                ))
        for c in out_copies:
            c.wait()

    return k(emo_id, strength, table_flat)


def kernel(emo_id, strength, table):
    tab65 = jnp.pad(table, ((0, 0), (0, _PITCH - EMO_DIM))).reshape(-1)
    w = _emotion_encode(emo_id.astype(jnp.int32), strength, tab65)
    return (
        w.reshape(_TD, 128, 8, 128)
        .transpose(1, 3, 0, 2)
        .reshape(BATCH, EMO_DIM)
    )


# speedup vs baseline: 2.1523x; 1.0978x over previous
import dataclasses
import functools

import jax
import jax.numpy as jnp
from jax import lax
from jax.experimental import pallas as pl
from jax.experimental.pallas import tpu as pltpu
from jax.experimental.pallas import tpu_sc as plsc

NUM_EMOTIONS = 1000
EMO_DIM = 64
_PITCH = EMO_DIM + 1
BATCH = 16384

_NC = 2
_NS = 16
_L = 16
_NW = _NC * _NS
_BPW = BATCH // _NW
_JPW = _BPW // 128
_TD = EMO_DIM // 8

_mesh = plsc.VectorSubcoreMesh(core_axis_name="c", subcore_axis_name="s")

_cp = pltpu.CompilerParams()
if "needs_layout_passes" in pltpu.CompilerParams.__dataclass_fields__:
    _cp = dataclasses.replace(_cp, needs_layout_passes=False)
if "use_tc_tiling_on_sc" in pltpu.CompilerParams.__dataclass_fields__:
    _cp = dataclasses.replace(_cp, use_tc_tiling_on_sc=False)


@jax.jit
def _emotion_encode(emo_id, strength, table_flat):
    @functools.partial(
        pl.kernel,
        out_type=jax.ShapeDtypeStruct((BATCH // 2, 2 * EMO_DIM), jnp.float32),
        mesh=_mesh,
        compiler_params=_cp,
        scratch_types=[
            pltpu.VMEM_SHARED((NUM_EMOTIONS * _PITCH,), jnp.float32),
            pltpu.VMEM((NUM_EMOTIONS * _PITCH,), jnp.float32),
            pltpu.VMEM((_BPW,), jnp.int32),
            pltpu.VMEM((_BPW,), jnp.float32),
        ]
        + [pltpu.VMEM((EMO_DIM, 128), jnp.float32) for _ in range(_JPW)]
        + [pltpu.SemaphoreType.DMA, pltpu.SemaphoreType.DMA],
    )
    def k(emo_hbm, str_hbm, tab_hbm, out_hbm, tab_s, tab_v, idx_v, str_v,
          t0, t1, t2, t3, sem_t, sem_o):
        sid = lax.axis_index("s")
        wid = sid * _NC + lax.axis_index("c")
        base = wid * _BPW

        @pl.when(sid == 0)
        def _():
            pltpu.sync_copy(tab_hbm, tab_s)

        pltpu.sync_copy(emo_hbm.at[pl.ds(base, _BPW)], idx_v)
        pltpu.sync_copy(str_hbm.at[pl.ds(base, _BPW)], str_v)
        plsc.subcore_barrier()
        pltpu.sync_copy(tab_s, tab_v)

        out_copies = []
        for jl, trows_v in enumerate((t0, t1, t2, t3)):
            @plsc.parallel_loop(0, 128 // _L)
            def _(bg, jl=jl, trows_v=trows_v):
                o = jl * 128 + bg * _L
                e = idx_v[pl.ds(o, _L)]
                s = str_v[pl.ds(o, _L)]
                ebase = e * _PITCH

                @plsc.parallel_loop(0, EMO_DIM, unroll=8)
                def _(d):
                    v = plsc.load_gather(tab_v, [ebase + d])
                    trows_v[d, pl.ds(bg * _L, _L)] = v * s

            jg = wid * _JPW + jl
            for i in range(_TD):
                out_copies.append(pltpu.async_copy(
                    trows_v.at[pl.ds(8 * i, 8)],
                    out_hbm.at[pl.ds((i * 128 + jg) * 8, 8)],
                    sem_o,
                ))
        for c in out_copies:
            c.wait()

    return k(emo_id, strength, table_flat)


def kernel(emo_id, strength, table):
    tab65 = jnp.pad(table, ((0, 0), (0, _PITCH - EMO_DIM))).reshape(-1)
    w = _emotion_encode(emo_id.astype(jnp.int32), strength, tab65)
    return (
        w.reshape(_TD, 128, 8, 128)
        .transpose(1, 3, 0, 2)
        .reshape(BATCH, EMO_DIM)
    )


# speedup vs baseline: 2.1609x; 1.0040x over previous
import dataclasses
import functools

import jax
import jax.numpy as jnp
from jax import lax
from jax.experimental import pallas as pl
from jax.experimental.pallas import tpu as pltpu
from jax.experimental.pallas import tpu_sc as plsc

NUM_EMOTIONS = 1000
EMO_DIM = 64
_PITCH = EMO_DIM + 1
BATCH = 16384

_NC = 2
_NS = 16
_L = 16
_NW = _NC * _NS
_BPW = BATCH // _NW
_JPW = _BPW // 128
_TD = EMO_DIM // 8

_mesh = plsc.VectorSubcoreMesh(core_axis_name="c", subcore_axis_name="s")

_cp = pltpu.CompilerParams()
if "needs_layout_passes" in pltpu.CompilerParams.__dataclass_fields__:
    _cp = dataclasses.replace(_cp, needs_layout_passes=False)
if "use_tc_tiling_on_sc" in pltpu.CompilerParams.__dataclass_fields__:
    _cp = dataclasses.replace(_cp, use_tc_tiling_on_sc=False)


@jax.jit
def _emotion_encode(emo_id, strength, table_flat):
    @functools.partial(
        pl.kernel,
        out_type=jax.ShapeDtypeStruct((BATCH // 2, 2 * EMO_DIM), jnp.float32),
        mesh=_mesh,
        compiler_params=_cp,
        scratch_types=[
            pltpu.VMEM_SHARED((NUM_EMOTIONS * _PITCH,), jnp.float32),
            pltpu.VMEM((NUM_EMOTIONS * _PITCH,), jnp.float32),
            pltpu.VMEM((_BPW,), jnp.int32),
            pltpu.VMEM((_BPW,), jnp.float32),
        ]
        + [pltpu.VMEM((EMO_DIM, 128), jnp.float32) for _ in range(2)]
        + [pltpu.SemaphoreType.DMA, pltpu.SemaphoreType.DMA],
    )
    def k(emo_hbm, str_hbm, tab_hbm, out_hbm, tab_s, tab_v, idx_v, str_v,
          t0, t1, sem_t, sem_o):
        sid = lax.axis_index("s")
        wid = sid * _NC + lax.axis_index("c")
        base = wid * _BPW

        @pl.when(sid == 0)
        def _():
            pltpu.sync_copy(tab_hbm, tab_s)

        pltpu.sync_copy(emo_hbm.at[pl.ds(base, _BPW)], idx_v)
        pltpu.sync_copy(str_hbm.at[pl.ds(base, _BPW)], str_v)
        plsc.subcore_barrier()
        pltpu.sync_copy(tab_s, tab_v)

        @pl.loop(0, _JPW // 2)
        def _(p):
            for half, trows_v in enumerate((t0, t1)):
                jl = 2 * p + half

                @plsc.parallel_loop(0, 128 // _L)
                def _(bg, jl=jl, trows_v=trows_v):
                    o = jl * 128 + bg * _L
                    e = idx_v[pl.ds(o, _L)]
                    s = str_v[pl.ds(o, _L)]
                    ebase = e * _PITCH

                    @plsc.parallel_loop(0, EMO_DIM, unroll=8)
                    def _(d):
                        v = plsc.load_gather(tab_v, [ebase + d])
                        trows_v[d, pl.ds(bg * _L, _L)] = v * s

                @pl.when(p > 0)
                def _(trows_v=trows_v):
                    for i in range(_TD):
                        pltpu.make_async_copy(
                            trows_v.at[pl.ds(8 * i, 8)],
                            out_hbm.at[pl.ds(i * 1024, 8)],
                            sem_o,
                        ).wait()

                jg = wid * _JPW + jl
                for i in range(_TD):
                    pltpu.async_copy(
                        trows_v.at[pl.ds(8 * i, 8)],
                        out_hbm.at[pl.ds((i * 128 + jg) * 8, 8)],
                        sem_o,
                    )

        for trows_v in (t0, t1):
            for i in range(_TD):
                pltpu.make_async_copy(
                    trows_v.at[pl.ds(8 * i, 8)],
                    out_hbm.at[pl.ds(i * 1024, 8)],
                    sem_o,
                ).wait()

    return k(emo_id, strength, table_flat)


def kernel(emo_id, strength, table):
    tab65 = jnp.pad(table, ((0, 0), (0, _PITCH - EMO_DIM))).reshape(-1)
    w = _emotion_encode(emo_id.astype(jnp.int32), strength, tab65)
    return (
        w.reshape(_TD, 128, 8, 128)
        .transpose(1, 3, 0, 2)
        .reshape(BATCH, EMO_DIM)
    )
